# baseline (speedup 1.0000x reference)
import functools

import jax
import jax.numpy as jnp
from jax import lax
from jax.experimental import pallas as pl
from jax.experimental.pallas import tpu as pltpu
from jax.experimental.pallas import tpu_sc as plsc

_VOCAB = 1000000
_DIM = 64
_BATCH = 16384

_INFO = plsc.get_sparse_core_info()
_NC = _INFO.num_cores
_NS = _INFO.num_subcores
_LANES = _INFO.num_lanes
_NW = _NC * _NS
_BPW = _BATCH // _NW
_CHUNK = 128
_NCHUNK = _BPW // _CHUNK
_GROUPS = _BPW // _LANES


def _sc_body(center_hbm, context_hbm, in_hbm, out_hbm, score_hbm,
             cidx_v, oidx_v, crows_v, orows_v, score_v, sem):
    wid = lax.axis_index("s") * _NC + lax.axis_index("c")
    base = wid * _BPW

    for j in range(_NCHUNK):
        pltpu.sync_copy(center_hbm.at[pl.ds(base + j * _CHUNK, _CHUNK)],
                        cidx_v.at[j])
        pltpu.sync_copy(context_hbm.at[pl.ds(base + j * _CHUNK, _CHUNK)],
                        oidx_v.at[j])

    copies = []
    for j in range(_NCHUNK):
        copies.append(pltpu.async_copy(
            in_hbm.at[cidx_v.at[j]],
            crows_v.at[pl.ds(j * _CHUNK, _CHUNK)], sem))
        copies.append(pltpu.async_copy(
            out_hbm.at[oidx_v.at[j]],
            orows_v.at[pl.ds(j * _CHUNK, _CHUNK)], sem))
    for c in copies:
        c.wait()

    lane = lax.iota(jnp.int32, _LANES)

    def group_body(g, carry):
        rows = g * _LANES + lane
        accs = [jnp.zeros((_LANES,), jnp.float32) for _ in range(4)]
        for d in range(_DIM):
            dcol = jnp.full((_LANES,), d, jnp.int32)
            c = plsc.load_gather(crows_v, [rows, dcol])
            o = plsc.load_gather(orows_v, [rows, dcol])
            accs[d % 4] = accs[d % 4] + c * o
        score_v[pl.ds(g * _LANES, _LANES)] = (
            (accs[0] + accs[1]) + (accs[2] + accs[3]))
        return carry

    lax.fori_loop(0, _GROUPS, group_body, 0)

    pltpu.sync_copy(score_v, score_hbm.at[pl.ds(base, _BPW)])


@functools.partial(
    pl.kernel,
    out_type=jax.ShapeDtypeStruct((_BATCH,), jnp.float32),
    mesh=plsc.VectorSubcoreMesh(core_axis_name="c", subcore_axis_name="s"),
    compiler_params=pltpu.CompilerParams(
        needs_layout_passes=False, use_tc_tiling_on_sc=False),
    scratch_types=[
        pltpu.VMEM((_NCHUNK, _CHUNK), jnp.int32),
        pltpu.VMEM((_NCHUNK, _CHUNK), jnp.int32),
        pltpu.VMEM((_BPW, _DIM), jnp.float32),
        pltpu.VMEM((_BPW, _DIM), jnp.float32),
        pltpu.VMEM((_BPW,), jnp.float32),
        pltpu.SemaphoreType.DMA,
    ],
)
def _w2v_score(center_hbm, context_hbm, in_hbm, out_hbm, score_hbm,
               cidx_v, oidx_v, crows_v, orows_v, score_v, sem):
    _sc_body(center_hbm, context_hbm, in_hbm, out_hbm, score_hbm,
             cidx_v, oidx_v, crows_v, orows_v, score_v, sem)


def kernel(center, context, in_embed, out_embed):
    return _w2v_score(center.astype(jnp.int32), context.astype(jnp.int32),
                      in_embed, out_embed)


# speedup vs baseline: 1.0023x; 1.0023x over previous
import functools

import jax
import jax.numpy as jnp
from jax import lax
from jax.experimental import pallas as pl
from jax.experimental.pallas import tpu as pltpu
from jax.experimental.pallas import tpu_sc as plsc

_VOCAB = 1000000
_DIM = 64
_BATCH = 16384
_PAIR = 128
_ROWS = _VOCAB * _DIM // _PAIR

_INFO = plsc.get_sparse_core_info()
_NC = _INFO.num_cores
_NS = _INFO.num_subcores
_LANES = _INFO.num_lanes
_NW = _NC * _NS
_BPW = _BATCH // _NW
_NQ = 4
_QL = _BPW // _NQ
_QG = _QL // _LANES


def _sc_body(center_hbm, context_hbm, in_hbm, out_hbm, score_hbm,
             cidx_v, oidx_v, pidx_v, cbuf_v, obuf_v, score_v, sem):
    wid = lax.axis_index("s") * _NC + lax.axis_index("c")
    base = wid * _BPW

    pltpu.sync_copy(center_hbm.at[pl.ds(base, _BPW)], cidx_v)
    pltpu.sync_copy(context_hbm.at[pl.ds(base, _BPW)], oidx_v)

    lane = lax.iota(jnp.int32, _LANES)

    def build_quarter(q, cslot, oslot):
        def g_body(g, carry):
            off = q * _QL + g * _LANES
            dst = g * _LANES
            pidx_v[cslot, pl.ds(dst, _LANES)] = cidx_v[pl.ds(off, _LANES)] >> 1
            pidx_v[oslot, pl.ds(dst, _LANES)] = oidx_v[pl.ds(off, _LANES)] >> 1
            return carry
        lax.fori_loop(0, _QG, g_body, 0)

    def fire_quarter(q, slot):
        cc = pltpu.async_copy(in_hbm.at[pidx_v.at[2 * slot]],
                              cbuf_v.at[slot], sem.at[slot])
        oc = pltpu.async_copy(out_hbm.at[pidx_v.at[2 * slot + 1]],
                              obuf_v.at[slot], sem.at[slot])
        return (cc, oc)

    def compute_quarter(q, slot):
        def g_body(g, carry):
            off = q * _QL + g * _LANES
            rows = g * _LANES + lane
            cpar = (cidx_v[pl.ds(off, _LANES)] & 1) << 6
            opar = (oidx_v[pl.ds(off, _LANES)] & 1) << 6
            accs = [jnp.zeros((_LANES,), jnp.float32) for _ in range(4)]
            ccol = cpar
            ocol = opar
            for d in range(_DIM):
                c = plsc.load_gather(cbuf_v.at[slot], [rows, ccol])
                o = plsc.load_gather(obuf_v.at[slot], [rows, ocol])
                accs[d % 4] = accs[d % 4] + c * o
                ccol = ccol + 1
                ocol = ocol + 1
            score_v[pl.ds(off, _LANES)] = (
                (accs[0] + accs[1]) + (accs[2] + accs[3]))
            return carry
        lax.fori_loop(0, _QG, g_body, 0)

    copies = [None] * _NQ
    for q in range(_NQ):
        if q >= 2:
            for c in copies[q - 2]:
                c.wait()
            compute_quarter(q - 2, (q - 2) % 2)
        build_quarter(q, (q % 2) * 2, (q % 2) * 2 + 1)
        copies[q] = fire_quarter(q, q % 2)
    for q in (_NQ - 2, _NQ - 1):
        for c in copies[q]:
            c.wait()
        compute_quarter(q, q % 2)

    pltpu.sync_copy(score_v, score_hbm.at[pl.ds(base, _BPW)])


@functools.partial(
    pl.kernel,
    out_type=jax.ShapeDtypeStruct((_BATCH,), jnp.float32),
    mesh=plsc.VectorSubcoreMesh(core_axis_name="c", subcore_axis_name="s"),
    compiler_params=pltpu.CompilerParams(
        needs_layout_passes=False,
        use_tc_tiling_on_sc=True,
    ),
    scratch_types=[
        pltpu.VMEM((_BPW,), jnp.int32),
        pltpu.VMEM((_BPW,), jnp.int32),
        pltpu.VMEM((4, _QL), jnp.int32),
        pltpu.VMEM((2, _QL, _PAIR), jnp.float32),
        pltpu.VMEM((2, _QL, _PAIR), jnp.float32),
        pltpu.VMEM((_BPW,), jnp.float32),
        pltpu.SemaphoreType.DMA((2,)),
    ],
)
def _w2v_score(center_hbm, context_hbm, in_hbm, out_hbm, score_hbm,
               cidx_v, oidx_v, pidx_v, cbuf_v, obuf_v, score_v, sem):
    _sc_body(center_hbm, context_hbm, in_hbm, out_hbm, score_hbm,
             cidx_v, oidx_v, pidx_v, cbuf_v, obuf_v, score_v, sem)


def kernel(center, context, in_embed, out_embed):
    in_p = in_embed.reshape(_ROWS, _PAIR)
    out_p = out_embed.reshape(_ROWS, _PAIR)
    return _w2v_score(center.astype(jnp.int32), context.astype(jnp.int32),
                      in_p, out_p)
